# baseline (speedup 1.0000x reference)
import functools

import jax
import jax.numpy as jnp
from jax import lax
from jax.experimental import pallas as pl
from jax.experimental.pallas import tpu as pltpu
from jax.experimental.pallas import tpu_sc as plsc

_N = 1048576
_NC = 2
_NS = 16
_NW = _NC * _NS
_PER_W = _N // _NW
_CHUNK = 4096
_NCHUNK = _PER_W // _CHUNK
_LANES = 16
_THRESHOLD = 0.01


def _chunk_accum(xb, yb, acc):
    def body(j, acc):
        xv = xb[pl.ds(j * _LANES, _LANES)]
        yv = yb[pl.ds(j * _LANES, _LANES)]
        d = jnp.abs(xv - yv)
        w = jnp.where(yv < _THRESHOLD, 4.0, 1.0).astype(jnp.float32)
        return acc + d * w
    return lax.fori_loop(0, _CHUNK // _LANES, body, acc)


@functools.partial(
    pl.kernel,
    mesh=plsc.VectorSubcoreMesh(core_axis_name="c", subcore_axis_name="s"),
    out_type=jax.ShapeDtypeStruct((_NW, _LANES), jnp.float32),
    scratch_types=[
        pltpu.VMEM((_CHUNK,), jnp.float32),
        pltpu.VMEM((_CHUNK,), jnp.float32),
        pltpu.VMEM((_CHUNK,), jnp.float32),
        pltpu.VMEM((_CHUNK,), jnp.float32),
        pltpu.VMEM((_LANES,), jnp.float32),
        pltpu.SemaphoreType.DMA,
        pltpu.SemaphoreType.DMA,
        pltpu.SemaphoreType.DMA,
        pltpu.SemaphoreType.DMA,
    ],
)
def _sc_partial_sums(x_hbm, y_hbm, out_hbm,
                     xb0, xb1, yb0, yb1, accv,
                     sx0, sx1, sy0, sy1):
    wid = lax.axis_index("s") * _NC + lax.axis_index("c")
    base = wid * _PER_W

    xbufs = (xb0, xb1)
    ybufs = (yb0, yb1)
    sxs = (sx0, sx1)
    sys_ = (sy0, sy1)

    def start(i, slot):
        src = pl.ds(base + i * _CHUNK, _CHUNK)
        cx = pltpu.async_copy(x_hbm.at[src], xbufs[slot], sxs[slot])
        cy = pltpu.async_copy(y_hbm.at[src], ybufs[slot], sys_[slot])
        return cx, cy

    acc = jnp.zeros((_LANES,), jnp.float32)
    pending = start(0, 0)
    for i in range(_NCHUNK):
        slot = i % 2
        nxt = None
        if i + 1 < _NCHUNK:
            nxt = start(i + 1, 1 - slot)
        pending[0].wait()
        pending[1].wait()
        acc = _chunk_accum(xbufs[slot], ybufs[slot], acc)
        pending = nxt

    accv[...] = acc
    pltpu.sync_copy(accv, out_hbm.at[wid])


def kernel(x, y):
    partials = _sc_partial_sums(x.reshape(_N), y.reshape(_N))
    loss = jnp.sum(partials) * (1.0 / _N)
    return loss.reshape(1, 1)


# speedup vs baseline: 1.0847x; 1.0847x over previous
import functools

import jax
import jax.numpy as jnp
from jax import lax
from jax.experimental import pallas as pl
from jax.experimental.pallas import tpu as pltpu
from jax.experimental.pallas import tpu_sc as plsc

_N = 1048576
_NC = 2
_NS = 16
_NW = _NC * _NS
_PER_W = _N // _NW
_CHUNK = 4096
_NCHUNK = _PER_W // _CHUNK
_LANES = 16
_THRESHOLD = 0.01


_UNROLL = 8


def _chunk_accum(xb, yb, accs):
    def body(j, accs):
        accs = list(accs)
        for u in range(_UNROLL):
            off = (j * _UNROLL + u) * _LANES
            xv = xb[pl.ds(off, _LANES)]
            yv = yb[pl.ds(off, _LANES)]
            d = jnp.abs(xv - yv)
            w = jnp.where(yv < _THRESHOLD, 4.0, 1.0).astype(jnp.float32)
            accs[u % 4] = accs[u % 4] + d * w
        return tuple(accs)
    return lax.fori_loop(0, _CHUNK // (_LANES * _UNROLL), body, accs)


@functools.partial(
    pl.kernel,
    mesh=plsc.VectorSubcoreMesh(core_axis_name="c", subcore_axis_name="s"),
    out_type=jax.ShapeDtypeStruct((_NW, _LANES), jnp.float32),
    scratch_types=[
        pltpu.VMEM((_CHUNK,), jnp.float32),
        pltpu.VMEM((_CHUNK,), jnp.float32),
        pltpu.VMEM((_CHUNK,), jnp.float32),
        pltpu.VMEM((_CHUNK,), jnp.float32),
        pltpu.VMEM((_LANES,), jnp.float32),
        pltpu.SemaphoreType.DMA,
        pltpu.SemaphoreType.DMA,
        pltpu.SemaphoreType.DMA,
        pltpu.SemaphoreType.DMA,
    ],
)
def _sc_partial_sums(x_hbm, y_hbm, out_hbm,
                     xb0, xb1, yb0, yb1, accv,
                     sx0, sx1, sy0, sy1):
    wid = lax.axis_index("s") * _NC + lax.axis_index("c")
    base = wid * _PER_W

    xbufs = (xb0, xb1)
    ybufs = (yb0, yb1)
    sxs = (sx0, sx1)
    sys_ = (sy0, sy1)

    def start(i, slot):
        src = pl.ds(base + i * _CHUNK, _CHUNK)
        cx = pltpu.async_copy(x_hbm.at[src], xbufs[slot], sxs[slot])
        cy = pltpu.async_copy(y_hbm.at[src], ybufs[slot], sys_[slot])
        return cx, cy

    zero = jnp.zeros((_LANES,), jnp.float32)
    accs = (zero, zero, zero, zero)
    pending = start(0, 0)
    for i in range(_NCHUNK):
        slot = i % 2
        nxt = None
        if i + 1 < _NCHUNK:
            nxt = start(i + 1, 1 - slot)
        pending[0].wait()
        pending[1].wait()
        accs = _chunk_accum(xbufs[slot], ybufs[slot], accs)
        pending = nxt

    accv[...] = (accs[0] + accs[1]) + (accs[2] + accs[3])
    pltpu.sync_copy(accv, out_hbm.at[wid])


def kernel(x, y):
    partials = _sc_partial_sums(x.reshape(_N), y.reshape(_N))
    loss = jnp.sum(partials) * (1.0 / _N)
    return loss.reshape(1, 1)
